# baseline (speedup 1.0000x reference)
import functools

import jax
import jax.numpy as jnp
from jax import lax
from jax.experimental import pallas as pl
from jax.experimental.pallas import tpu as pltpu
from jax.experimental.pallas import tpu_sc as plsc

N_REGS = 32
BIT_WIDTH = 64
KEY_DIM = 128
BATCH = 16384

_NC = 2
_NS = 16
_NW = _NC * _NS
_BPW = BATCH // _NW
_CB = 128
_NCB = _BPW // _CB
_L = 16


def _table_body(temp_ref, keys_ref, w1_ref, b1_ref, w2_ref, b2_ref, vals_ref,
                out_ref):
    r = lax.broadcasted_iota(jnp.int32, (N_REGS, 8), 0)
    c = lax.broadcasted_iota(jnp.int32, (N_REGS, 8), 1)
    bits = jnp.where(c < 5, (r >> c) & 1, 0).astype(jnp.float32)
    w1 = jnp.where(lax.broadcasted_iota(jnp.int32, (8, KEY_DIM), 0) < 5,
                   w1_ref[...], 0.0)
    h = jnp.dot(bits, w1, preferred_element_type=jnp.float32) + b1_ref[...]
    h = 0.5 * h * (1.0 + lax.erf(h * (2.0 ** -0.5)))
    q = jnp.dot(h, w2_ref[...], preferred_element_type=jnp.float32) + b2_ref[...]
    sim = lax.dot_general(q, keys_ref[...], (((1,), (1,)), ((), ())),
                          preferred_element_type=jnp.float32)
    inv_temp = 1.0 / jnp.maximum(jnp.abs(temp_ref[0]), 0.1)
    sim = sim * inv_temp
    m = jnp.max(sim, axis=1, keepdims=True)
    e = jnp.exp(sim - m)
    attn = e / jnp.sum(e, axis=1, keepdims=True)
    tab = jnp.dot(attn, vals_ref[...], preferred_element_type=jnp.float32)
    row = lax.broadcasted_iota(jnp.int32, (N_REGS, BIT_WIDTH), 0)
    tab = jnp.where(row == N_REGS - 1, 0.0, tab)
    out_ref[...] = jnp.concatenate(
        [tab, jnp.zeros((N_REGS, BIT_WIDTH), jnp.float32)], axis=1)


def _build_table(temperature, keys, w1, b1, w2, b2, vals):
    return pl.pallas_call(
        _table_body,
        out_shape=jax.ShapeDtypeStruct((N_REGS, 2 * BIT_WIDTH), jnp.float32),
        in_specs=[pl.BlockSpec(memory_space=pltpu.SMEM)] +
                 [pl.BlockSpec(memory_space=pltpu.VMEM)] * 6,
    )(temperature, keys, w1, b1, w2, b2, vals)


@functools.cache
def _gather_kernel():
    mesh = plsc.VectorSubcoreMesh(core_axis_name="c", subcore_axis_name="s")

    @functools.partial(
        pl.kernel,
        mesh=mesh,
        compiler_params=pltpu.CompilerParams(needs_layout_passes=False),
        out_type=jax.ShapeDtypeStruct((BIT_WIDTH, BATCH), jnp.float32),
        scratch_types=[
            pltpu.VMEM((_BPW,), jnp.int32),
            pltpu.VMEM((N_REGS * 2 * BIT_WIDTH,), jnp.float32),
            [pltpu.VMEM((BIT_WIDTH, _CB), jnp.float32) for _ in range(_NCB)],
            pltpu.SemaphoreType.DMA,
        ],
    )
    def _gather(table_hbm, idx_hbm, out_hbm, idx_v, table_v, bufs, sem):
        wid = lax.axis_index("s") * _NC + lax.axis_index("c")
        base = wid * _BPW
        pltpu.sync_copy(table_hbm, table_v)
        pltpu.sync_copy(idx_hbm.at[pl.ds(base, _BPW)], idx_v)

        copies = []
        for cb in range(_NCB):
            buf = bufs[cb]

            def _group(kg, cb=cb, buf=buf):
                idx16 = idx_v[pl.ds(cb * _CB + kg * _L, _L)]
                flat16 = idx16 * (2 * BIT_WIDTH)
                for d in range(BIT_WIDTH):
                    buf[d, pl.ds(kg * _L, _L)] = plsc.load_gather(
                        table_v, [flat16 + d])

            plsc.parallel_loop(0, _CB // _L, unroll=4)(_group)
            copies.append(pltpu.async_copy(
                buf, out_hbm.at[:, pl.ds(base + cb * _CB, _CB)], sem))
        for cp in copies:
            cp.wait()

    return _gather


def kernel(idx, register_keys, W1, b1, W2, b2, temperature, register_values):
    w1p = jnp.concatenate([W1, W1[:3]], axis=0)
    table = _build_table(temperature.reshape(1), register_keys, w1p,
                         b1.reshape(1, KEY_DIM), W2, b2.reshape(1, KEY_DIM),
                         register_values)
    out_t = _gather_kernel()(table.reshape(-1), idx.astype(jnp.int32))
    return out_t.T


# speedup vs baseline: 1.4453x; 1.4453x over previous
import functools

import jax
import jax.numpy as jnp
from jax import lax
from jax.experimental import pallas as pl
from jax.experimental.pallas import tpu as pltpu
from jax.experimental.pallas import tpu_sc as plsc

N_REGS = 32
BIT_WIDTH = 64
KEY_DIM = 128
BATCH = 16384

_NC = 2
_NS = 16
_NW = _NC * _NS
_BPW = BATCH // _NW
_CB = 128
_NCB = _BPW // _CB
_L = 16
_STRIDE = 65


def _table_body(temp_ref, keys_ref, w1_ref, b1_ref, w2_ref, b2_ref, vals_ref,
                out_ref):
    r = lax.broadcasted_iota(jnp.int32, (N_REGS, 8), 0)
    c = lax.broadcasted_iota(jnp.int32, (N_REGS, 8), 1)
    bits = jnp.where(c < 5, (r >> c) & 1, 0).astype(jnp.float32)
    w1 = jnp.where(lax.broadcasted_iota(jnp.int32, (8, KEY_DIM), 0) < 5,
                   w1_ref[...], 0.0)
    h = jnp.dot(bits, w1, preferred_element_type=jnp.float32) + b1_ref[...]
    h = 0.5 * h * (1.0 + lax.erf(h * (2.0 ** -0.5)))
    q = jnp.dot(h, w2_ref[...], preferred_element_type=jnp.float32) + b2_ref[...]
    sim = lax.dot_general(q, keys_ref[...], (((1,), (1,)), ((), ())),
                          preferred_element_type=jnp.float32)
    inv_temp = 1.0 / jnp.maximum(jnp.abs(temp_ref[0]), 0.1)
    sim = sim * inv_temp
    m = jnp.max(sim, axis=1, keepdims=True)
    e = jnp.exp(sim - m)
    attn = e / jnp.sum(e, axis=1, keepdims=True)
    tab = jnp.dot(attn, vals_ref[...], preferred_element_type=jnp.float32)
    row = lax.broadcasted_iota(jnp.int32, (N_REGS, BIT_WIDTH), 0)
    tab = jnp.where(row == N_REGS - 1, 0.0, tab)
    out_ref[...] = jnp.concatenate(
        [tab, jnp.zeros((N_REGS, BIT_WIDTH), jnp.float32)], axis=1)


def _build_table(temperature, keys, w1, b1, w2, b2, vals):
    return pl.pallas_call(
        _table_body,
        out_shape=jax.ShapeDtypeStruct((N_REGS, 2 * BIT_WIDTH), jnp.float32),
        in_specs=[pl.BlockSpec(memory_space=pltpu.SMEM)] +
                 [pl.BlockSpec(memory_space=pltpu.VMEM)] * 6,
    )(temperature, keys, w1, b1, w2, b2, vals)


@functools.cache
def _gather_kernel():
    mesh = plsc.VectorSubcoreMesh(core_axis_name="c", subcore_axis_name="s")

    @functools.partial(
        pl.kernel,
        mesh=mesh,
        compiler_params=pltpu.CompilerParams(needs_layout_passes=False),
        out_type=jax.ShapeDtypeStruct((BIT_WIDTH, BATCH), jnp.float32),
        scratch_types=[
            pltpu.VMEM((_BPW,), jnp.int32),
            pltpu.VMEM((N_REGS * 2 * BIT_WIDTH,), jnp.float32),
            pltpu.VMEM((N_REGS * _STRIDE,), jnp.float32),
            [pltpu.VMEM((BIT_WIDTH, _CB), jnp.float32) for _ in range(_NCB)],
            pltpu.SemaphoreType.DMA,
        ],
    )
    def _gather(table_hbm, idx_hbm, out_hbm, idx_v, table_v, table_s, bufs,
                sem):
        wid = lax.axis_index("s") * _NC + lax.axis_index("c")
        base = wid * _BPW
        pltpu.sync_copy(table_hbm, table_v)
        pltpu.sync_copy(idx_hbm.at[pl.ds(base, _BPW)], idx_v)

        def _restage(r, carry):
            for c8 in range(BIT_WIDTH // _L):
                table_s[pl.ds(r * _STRIDE + c8 * _L, _L)] = (
                    table_v[pl.ds(r * 2 * BIT_WIDTH + c8 * _L, _L)])
            return carry

        lax.fori_loop(0, N_REGS, _restage, 0)

        copies = []
        for cb in range(_NCB):
            buf = bufs[cb]

            def _group(kg, cb=cb, buf=buf):
                idx16 = idx_v[pl.ds(cb * _CB + kg * _L, _L)]
                flat16 = idx16 * _STRIDE
                for d in range(BIT_WIDTH):
                    buf[d, pl.ds(kg * _L, _L)] = plsc.load_gather(
                        table_s, [flat16 + d])

            plsc.parallel_loop(0, _CB // _L)(_group)
            copies.append(pltpu.async_copy(
                buf, out_hbm.at[:, pl.ds(base + cb * _CB, _CB)], sem))
        for cp in copies:
            cp.wait()

    return _gather


def kernel(idx, register_keys, W1, b1, W2, b2, temperature, register_values):
    w1p = jnp.concatenate([W1, W1[:3]], axis=0)
    table = _build_table(temperature.reshape(1), register_keys, w1p,
                         b1.reshape(1, KEY_DIM), W2, b2.reshape(1, KEY_DIM),
                         register_values)
    out_t = _gather_kernel()(table.reshape(-1), idx.astype(jnp.int32))
    return out_t.T
